# Initial kernel scaffold; baseline (speedup 1.0000x reference)
import jax
import jax.numpy as jnp
from jax.experimental import pallas as pl


def kernel(boxes, scores, boxes_prev, scores_prev):
    raise NotImplementedError("write your pallas kernel here")


# speedup vs baseline: 48.6609x; 48.6609x over previous
import functools

import jax
import jax.numpy as jnp
from jax import lax
from jax.experimental import pallas as pl
from jax.experimental.pallas import tpu as pltpu
from jax.experimental.pallas import tpu_sc as plsc

H = 512
W = 512
N_CUR = 5000
M_PREV = 5000
NP = 5120
SUB = 2
BLK = 32
GH = 16
GW = 16
HS = 256
WS = 256
MB = 256
NSTEPS = NP // MB
NWORK = 32
PERW = NP // NWORK
CHUNK = 8
BIGI = 2 ** 30


def _iou_body(c_ref, p_ref, biou_ref, bj_ref):
    k = pl.program_id(0)
    ax1 = c_ref[0:1, :]
    ay1 = c_ref[1:2, :]
    ax2 = c_ref[2:3, :]
    ay2 = c_ref[3:4, :]
    pb = p_ref[...]
    bx1 = pb[:, 0:1]
    by1 = pb[:, 1:2]
    bx2 = pb[:, 2:3]
    by2 = pb[:, 3:4]
    xl = jnp.maximum(ax1, bx1)
    yt = jnp.maximum(ay1, by1)
    xr = jnp.minimum(ax2, bx2)
    yb = jnp.minimum(ay2, by2)
    inter = jnp.maximum(xr - xl, 0.0) * jnp.maximum(yb - yt, 0.0)
    aa = (ax2 - ax1) * (ay2 - ay1)
    bb = (bx2 - bx1) * (by2 - by1)
    iou = inter / (aa + bb - inter)
    blk_best = jnp.max(iou, axis=0, keepdims=True)
    rowid = lax.broadcasted_iota(jnp.int32, iou.shape, 0) + k * MB
    blk_j = jnp.min(jnp.where(iou == blk_best, rowid, BIGI), axis=0,
                    keepdims=True)

    @pl.when(k == 0)
    def _():
        biou_ref[0:1, :] = blk_best
        bj_ref[0:1, :] = blk_j

    @pl.when(k > 0)
    def _():
        run = biou_ref[0:1, :]
        better = blk_best > run
        biou_ref[0:1, :] = jnp.where(better, blk_best, run)
        bj_ref[0:1, :] = jnp.where(better, blk_j, bj_ref[0:1, :])


def _stage1(bsub_t, psub128):
    return pl.pallas_call(
        _iou_body,
        grid=(NSTEPS,),
        in_specs=[
            pl.BlockSpec((8, NP), lambda k: (0, 0)),
            pl.BlockSpec((MB, 128), lambda k: (k, 0)),
        ],
        out_specs=[
            pl.BlockSpec((8, NP), lambda k: (0, 0)),
            pl.BlockSpec((8, NP), lambda k: (0, 0)),
        ],
        out_shape=[
            jax.ShapeDtypeStruct((8, NP), jnp.float32),
            jax.ShapeDtypeStruct((8, NP), jnp.int32),
        ],
    )(bsub_t, psub128)


_SC_TABS = 9


def _sc_body(bj_hbm, biou_hbm, tab_hbm, gout_hbm, cnt_hbm, *scratch):
    tabs = scratch[0:_SC_TABS]
    gbufs = scratch[_SC_TABS:2 * _SC_TABS]
    bjf, biouf, cnt_v = scratch[2 * _SC_TABS:]
    w = lax.axis_index("s") * 2 + lax.axis_index("c")
    base = w * PERW
    pltpu.sync_copy(bj_hbm, bjf)
    pltpu.sync_copy(biou_hbm, biouf)
    for r in range(_SC_TABS):
        pltpu.sync_copy(tab_hbm.at[pl.ds(r * NP, NP)], tabs[r])
    for c in range(PERW // 16):
        idx = bjf[pl.ds(base + c * 16, 16)]
        for r in range(_SC_TABS):
            gbufs[r][pl.ds(c * 16, 16)] = plsc.load_gather(tabs[r], [idx])
    for r in range(_SC_TABS):
        pltpu.sync_copy(gbufs[r], gout_hbm.at[pl.ds(r * NP + base, PERW)])
    for c in range(PERW // 16):
        cnt_v[pl.ds(c * 16, 16)] = jnp.zeros((16,), jnp.float32)
    ones = jnp.ones((16,), jnp.float32)

    def body(c, carry):
        idx = bjf[pl.ds(c * 16, 16)]
        m = biouf[pl.ds(c * 16, 16)] > 0.0
        il = idx - base
        inr = m & (il >= 0) & (il < PERW)
        ilc = jnp.clip(il, 0, PERW - 1)
        plsc.store_scatter(cnt_v, [ilc], ones, mask=inr)
        return carry

    lax.fori_loop(0, NP // 16, body, 0)
    pltpu.sync_copy(cnt_v, cnt_hbm.at[pl.ds(base, PERW)])


def _stage2(bj, biou, tab_flat):
    mesh = plsc.VectorSubcoreMesh(core_axis_name="c", subcore_axis_name="s")
    scr = ([pltpu.VMEM((NP,), jnp.float32) for _ in range(_SC_TABS)]
           + [pltpu.VMEM((PERW,), jnp.float32) for _ in range(_SC_TABS)]
           + [pltpu.VMEM((NP,), jnp.int32),
              pltpu.VMEM((NP,), jnp.float32),
              pltpu.VMEM((PERW,), jnp.float32)])
    fn = functools.partial(
        pl.kernel, mesh=mesh,
        out_type=[jax.ShapeDtypeStruct((_SC_TABS * NP,), jnp.float32),
                  jax.ShapeDtypeStruct((NP,), jnp.float32)],
        scratch_types=scr,
        compiler_params=pltpu.CompilerParams(needs_layout_passes=False),
    )(_sc_body)
    return fn(bj, biou, tab_flat)


def _ind(lo, hi, lane):
    return jnp.where((lane >= lo) & (lane <= hi) & (lane < GH), 1.0, 0.0)


def _dot_t(a, b):
    return lax.dot_general(a, b, (((0,), (0,)), ((), ())),
                           precision=lax.Precision.HIGHEST,
                           preferred_element_type=jnp.float32)


def _mm(a, b):
    return lax.dot_general(a, b, (((1,), (0,)), ((), ())),
                           precision=lax.Precision.HIGHEST,
                           preferred_element_type=jnp.float32)


def _paint_grid_body(items_ref, opsc_ref, bp_ref, mask_ref, grid_ref):
    f32 = jnp.float32
    big = f32(1e9)
    mask_ref[...] = jnp.zeros((HS, WS), f32)
    lane = lax.broadcasted_iota(jnp.int32, (1, WS), 1).astype(f32)

    def paint_group(g0, vfn):
        def chunk(c, carry):
            blk = items_ref[pl.ds(g0 + c * CHUNK, CHUNK), :]
            x1 = blk[:, 0:1]
            y1 = blk[:, 1:2]
            x2 = blk[:, 2:3]
            y2 = blk[:, 3:4]
            v = vfn(blk[:, 4:5], blk[:, 5:6])
            yv = jnp.where((lane >= y1) & (lane < y2), v, 0.0)
            xb = jnp.where((lane >= x1) & (lane < x2), big, 0.0)
            yvt = yv.T
            m = mask_ref[...]
            for kk in range(CHUNK):
                m = jnp.maximum(
                    m, jnp.minimum(yvt[:, kk:kk + 1], xb[kk:kk + 1, :]))
            mask_ref[...] = m
            return carry

        lax.fori_loop(0, NP // CHUNK, chunk, 0)

    paint_group(0, lambda a, b: (1.0 - a) * b)
    paint_group(NP, lambda a, b: jnp.where(a > 0.0, (1.0 - a) * b, 0.0))
    paint_group(2 * NP, lambda a, b: jnp.where(a < 0.5, b, 0.0))

    bc = opsc_ref[...]
    bp = bp_ref[...]
    lane = lax.broadcasted_iota(jnp.int32, (1, 128), 1).astype(f32)

    def cell_rng(x1, x2):
        return jnp.floor(x1 / BLK), jnp.floor((x2 - 1.0) / BLK)

    cx1, cx2 = cell_rng(bc[:, 0:1], bc[:, 2:3])
    cy1, cy2 = cell_rng(bc[:, 1:2], bc[:, 3:4])
    px1, px2 = cell_rng(bp[:, 0:1], bp[:, 2:3])
    py1, py2 = cell_rng(bp[:, 1:2], bp[:, 3:4])
    yc = _ind(cy1, cy2, lane)
    xc = _ind(cx1, cx2, lane)
    yp = _ind(py1, py2, lane)
    xp = _ind(px1, px2, lane)
    occ = _dot_t(yc, xc) + _dot_t(yp, xp)

    biou = bc[:, 5:6]
    matched = biou > 0.0
    mf = jnp.where(matched, 1.0, 0.0)
    mx1, mx2 = cell_rng(bc[:, 6:7], bc[:, 8:9])
    my1, my2 = cell_rng(bc[:, 7:8], bc[:, 9:10])
    ym = _ind(my1, my2, lane) * mf
    xm = _ind(mx1, mx2, lane) * mf

    def bilin(y, x):
        return jnp.sum(_mm(y, occ) * x, axis=1, keepdims=True)

    area_c = (jnp.sum(yc, axis=1, keepdims=True)
              * jnp.sum(xc, axis=1, keepdims=True))
    area_m = (jnp.sum(ym, axis=1, keepdims=True)
              * jnp.sum(xm, axis=1, keepdims=True))
    s = bilin(yc, xc) + bilin(ym, xm) - bilin(yc * ym, xc * xm) \
        - area_c - area_m
    isolated = s < 0.5
    h_c = bc[:, 3:4] - bc[:, 1:2]
    bigbox = isolated & (h_c >= 100.0) & (bc[:, 4:5] >= 0.7)
    bigf = jnp.where(bigbox, 1.0, 0.0)

    ux1 = jnp.where(matched, jnp.minimum(bc[:, 0:1], bc[:, 6:7]), bc[:, 0:1])
    uy1 = jnp.where(matched, jnp.minimum(bc[:, 1:2], bc[:, 7:8]), bc[:, 1:2])
    ux2 = jnp.where(matched, jnp.maximum(bc[:, 2:3], bc[:, 8:9]), bc[:, 2:3])
    uy2 = jnp.where(matched, jnp.maximum(bc[:, 3:4], bc[:, 9:10]), bc[:, 3:4])
    gx1, gx2 = cell_rng(ux1, ux2)
    gy1, gy2 = cell_rng(uy1, uy2)
    yu = _ind(gy1, gy2, lane)
    xu = _ind(gx1, gx2, lane)
    cnt2 = _dot_t(yu * bigf, xu)
    cnta = _dot_t(yu, xu)
    grid_ref[...] = jnp.where(cnt2 > 0.0, 2.0,
                              jnp.where(cnta > 0.0, 1.0, 0.0))[0:16, 0:128]


def _stage3(items, opsc, bp128):
    return pl.pallas_call(
        _paint_grid_body,
        out_shape=[
            jax.ShapeDtypeStruct((HS, WS), jnp.float32),
            jax.ShapeDtypeStruct((16, 128), jnp.float32),
        ],
    )(items, opsc, bp128)


def _pad_boxes(a, x2off=0.0):
    pad = jnp.tile(
        jnp.array([[600.0, 600.0, 600.0 + x2off, 600.0 + x2off]],
                  jnp.float32), (NP - a.shape[0], 1))
    return jnp.concatenate([a.astype(jnp.float32), pad], axis=0)


def _pad_vec(a):
    return jnp.concatenate(
        [a.astype(jnp.float32), jnp.zeros((NP - a.shape[0],), jnp.float32)])


def kernel(boxes, scores, boxes_prev, scores_prev):
    f32 = jnp.float32
    bsub_p = _pad_boxes(jnp.floor(boxes / SUB))
    psub_p = _pad_boxes(jnp.floor(boxes_prev / SUB), x2off=1.0)
    boxes_p = _pad_boxes(boxes)
    bp_p = _pad_boxes(boxes_prev, x2off=1.0)
    sc_p = _pad_vec(scores)
    sp_p = _pad_vec(scores_prev)

    bsub_t = jnp.zeros((8, NP), f32).at[0:4, :].set(bsub_p.T)
    psub128 = jnp.zeros((NP, 128), f32).at[:, 0:4].set(psub_p)

    biou8, bj8 = _stage1(bsub_t, psub128)
    biou = biou8[0]
    bj = bj8[0]

    tab = jnp.concatenate([psub_p.T, bp_p.T, sp_p[None, :]], axis=0)
    tab_flat = tab.reshape(-1)
    gout_flat, cnt = _stage2(bj, biou, tab_flat)
    gout = gout_flat.reshape(_SC_TABS, NP)
    psub_m = gout[0:4]
    pbm = gout[4:8]
    sp_m = gout[8]

    items = jnp.zeros((3 * NP, 128), f32)
    items = items.at[0:NP, 0:4].set(bsub_p)
    items = items.at[0:NP, 4].set(biou)
    items = items.at[0:NP, 5].set(sc_p)
    items = items.at[NP:2 * NP, 0:4].set(psub_m.T)
    items = items.at[NP:2 * NP, 4].set(biou)
    items = items.at[NP:2 * NP, 5].set(sp_m)
    items = items.at[2 * NP:, 0:4].set(psub_p)
    items = items.at[2 * NP:, 4].set(cnt)
    items = items.at[2 * NP:, 5].set(sp_p)

    opsc = jnp.zeros((NP, 128), f32)
    opsc = opsc.at[:, 0:4].set(boxes_p)
    opsc = opsc.at[:, 4].set(sc_p)
    opsc = opsc.at[:, 5].set(biou)
    opsc = opsc.at[:, 6:10].set(pbm.T)
    bp128 = jnp.zeros((NP, 128), f32).at[:, 0:4].set(bp_p)

    mask, gridp = _stage3(items, opsc, bp128)
    return mask[None, None, :, :], gridp[:, 0:16][None, None, :, :]
